# baseline (speedup 1.0000x reference)
import jax
import jax.numpy as jnp
from jax import lax
from jax.experimental import pallas as pl
from jax.experimental.pallas import tpu as pltpu

NEG_SLOPE = 0.2
BN_EPS = 1e-5
BBA = 32
BBB = 32
NPAD = 128
ROWS = 56


def _leaky(x):
    return jnp.where(x >= 0, x, NEG_SLOPE * x)


def _kern_a(x_ref, wcat_ref, btr_ref, adjn_ref, ws_ref, wg_ref,
            bg_ref, wlast_ref, out1_ref, h_ref, y_ref, o2a_ref, xb_scr,
            vt_scr, t_scr, *, n_nodes, d, hw):
    cf = x_ref.shape[2]
    for i in range(BBA):
        xb_scr[i * ROWS:i * ROWS + hw, :] = x_ref[i]
        xb_scr[i * ROWS + hw:(i + 1) * ROWS, :] = \
            jnp.zeros((ROWS - hw, cf), jnp.bfloat16)
    xb = xb_scr[...]
    s_all = jnp.dot(xb, wcat_ref[...],
                    preferred_element_type=jnp.float32)
    sc_all = s_all[:, :n_nodes]
    xt_all = s_all[:, NPAD:] + btr_ref[...]
    rows = ROWS
    valid = lax.broadcasted_iota(jnp.int32, (rows, 1), 0) < hw
    for i in range(BBA):
        sc = sc_all[i * rows:(i + 1) * rows, :]
        out1_ref[i:i + 1, :] = jnp.max(
            jnp.where(valid, sc, -jnp.inf), axis=0, keepdims=True)
        mask = jnp.where(valid, jax.nn.sigmoid(sc), 0.0)
        xt = xt_all[i * rows:(i + 1) * rows, :]
        vt_i = lax.dot_general(mask.astype(jnp.bfloat16),
                               xt.astype(jnp.bfloat16),
                               (((0,), (0,)), ((), ())),
                               preferred_element_type=jnp.float32)
        vt_scr[i * n_nodes:(i + 1) * n_nodes, :] = \
            vt_i.astype(jnp.bfloat16)
        o2a_ref[i * n_nodes:(i + 1) * n_nodes, :] = \
            jnp.sum(wlast_ref[...] * vt_i, axis=1, keepdims=True)
    v_bf = vt_scr[...]
    for i in range(BBA):
        t_i = jnp.dot(adjn_ref[...],
                      v_bf[i * n_nodes:(i + 1) * n_nodes, :],
                      preferred_element_type=jnp.float32)
        t_scr[i * n_nodes:(i + 1) * n_nodes, :] = \
            _leaky(t_i).astype(jnp.bfloat16)
    h_all = v_bf.astype(jnp.float32) + jnp.dot(
        t_scr[...], ws_ref[...], preferred_element_type=jnp.float32)
    h_ref[...] = h_all.astype(jnp.bfloat16)
    xglb = jnp.mean(h_all.reshape(BBA, n_nodes, d), axis=1)
    y_ref[...] = lax.dot_general(xglb.astype(jnp.bfloat16), wg_ref[...],
                                 (((1,), (1,)), ((), ())),
                                 preferred_element_type=jnp.float32) \
        + bg_ref[...]


def _kern_b(yfull_ref, yblk_ref, gamma_ref, beta_ref, h_ref, wcog_ref,
            wcox_ref, bco_ref, sadj_ref, wdyn_ref, wlast_ref, out2_ref,
            t_scr, *, n_nodes, d):
    yf = yfull_ref[...]
    mu = jnp.mean(yf, axis=0, keepdims=True)
    var = jnp.mean((yf - mu) ** 2, axis=0, keepdims=True)
    g = _leaky((yblk_ref[...] - mu) * lax.rsqrt(var + BN_EPS)
               * gamma_ref[...] + beta_ref[...])
    tg = lax.dot_general(wcog_ref[...], g.astype(jnp.bfloat16),
                         (((1,), (1,)), ((), ())),
                         preferred_element_type=jnp.float32)
    tx_all = lax.dot_general(wcox_ref[...], h_ref[...],
                             (((1,), (1,)), ((), ())),
                             preferred_element_type=jnp.float32)
    for i in range(BBB):
        h_bf = h_ref[i * n_nodes:(i + 1) * n_nodes, :]
        tx = tx_all[:, i * n_nodes:(i + 1) * n_nodes]
        a = jax.nn.sigmoid(tx + tg[:, i:i + 1] + bco_ref[...])
        a = (a + sadj_ref[...]) * 0.5
        dv = lax.rsqrt(jnp.sum(a, axis=1, keepdims=True))
        m = (dv * h_bf.astype(jnp.float32)).astype(jnp.bfloat16)
        t_i = lax.dot_general(a.astype(jnp.bfloat16), m,
                              (((0,), (0,)), ((), ())),
                              preferred_element_type=jnp.float32)
        t_scr[i * n_nodes:(i + 1) * n_nodes, :] = \
            _leaky(dv * t_i).astype(jnp.bfloat16)
    z = _leaky(jnp.dot(t_scr[...], wdyn_ref[...],
                       preferred_element_type=jnp.float32))
    zw = z.reshape(BBB, n_nodes, d) * wlast_ref[...][None]
    out2_ref[...] = jnp.sum(zw, axis=2)


def kernel(x_feat, static_adj, static_weight, dynamic_weight, w_fc, w_tr,
           b_tr, w_g, b_g, bn_gamma, bn_beta, w_co, b_co, w_last, b_last):
    B, Cf, H, W = x_feat.shape
    n_nodes = w_fc.shape[0]
    d = w_tr.shape[0]
    hw = H * W

    xt2 = x_feat.reshape(B, Cf, hw).astype(jnp.bfloat16)
    xt2 = xt2.transpose(0, 2, 1)
    wcat_t = jnp.concatenate(
        [w_fc.T.astype(jnp.bfloat16),
         jnp.zeros((Cf, NPAD - n_nodes), jnp.bfloat16),
         w_tr.T.astype(jnp.bfloat16)], axis=1)
    A = static_adj
    dvec = jnp.sum(A, axis=1) ** -0.5
    adjn = (dvec[:, None] * A.T * dvec[None, :]).astype(jnp.bfloat16)

    out1, h, y, o2a = pl.pallas_call(
        lambda *refs: _kern_a(*refs, n_nodes=n_nodes, d=d, hw=hw),
        grid=(B // BBA,),
        in_specs=[
            pl.BlockSpec((BBA, hw, Cf), lambda i: (i, 0, 0)),
            pl.BlockSpec((Cf, NPAD + d), lambda i: (0, 0)),
            pl.BlockSpec((1, d), lambda i: (0, 0)),
            pl.BlockSpec((n_nodes, n_nodes), lambda i: (0, 0)),
            pl.BlockSpec((d, d), lambda i: (0, 0)),
            pl.BlockSpec((d, d), lambda i: (0, 0)),
            pl.BlockSpec((1, d), lambda i: (0, 0)),
            pl.BlockSpec((n_nodes, d), lambda i: (0, 0)),
        ],
        out_specs=[
            pl.BlockSpec((BBA, n_nodes), lambda i: (i, 0)),
            pl.BlockSpec((BBA * n_nodes, d), lambda i: (i, 0)),
            pl.BlockSpec((BBA, d), lambda i: (i, 0)),
            pl.BlockSpec((BBA * n_nodes, 1), lambda i: (i, 0)),
        ],
        out_shape=[
            jax.ShapeDtypeStruct((B, n_nodes), jnp.float32),
            jax.ShapeDtypeStruct((B * n_nodes, d), jnp.bfloat16),
            jax.ShapeDtypeStruct((B, d), jnp.float32),
            jax.ShapeDtypeStruct((B * n_nodes, 1), jnp.float32),
        ],
        scratch_shapes=[pltpu.VMEM((BBA * ROWS, Cf), jnp.bfloat16),
                        pltpu.VMEM((BBA * n_nodes, d), jnp.bfloat16),
                        pltpu.VMEM((BBA * n_nodes, d), jnp.bfloat16)],
        compiler_params=pltpu.CompilerParams(
            dimension_semantics=("parallel",)),
    )(xt2, wcat_t, b_tr.reshape(1, d), adjn,
      static_weight.astype(jnp.bfloat16), w_g.astype(jnp.bfloat16),
      b_g.reshape(1, d), w_last)

    out2b = pl.pallas_call(
        lambda *refs: _kern_b(*refs, n_nodes=n_nodes, d=d),
        grid=(B // BBB,),
        in_specs=[
            pl.BlockSpec((B, d), lambda i: (0, 0)),
            pl.BlockSpec((BBB, d), lambda i: (i, 0)),
            pl.BlockSpec((1, d), lambda i: (0, 0)),
            pl.BlockSpec((1, d), lambda i: (0, 0)),
            pl.BlockSpec((BBB * n_nodes, d), lambda i: (i, 0)),
            pl.BlockSpec((n_nodes, d), lambda i: (0, 0)),
            pl.BlockSpec((n_nodes, d), lambda i: (0, 0)),
            pl.BlockSpec((n_nodes, 1), lambda i: (0, 0)),
            pl.BlockSpec((n_nodes, n_nodes), lambda i: (0, 0)),
            pl.BlockSpec((d, d), lambda i: (0, 0)),
            pl.BlockSpec((n_nodes, d), lambda i: (0, 0)),
        ],
        out_specs=pl.BlockSpec((BBB, n_nodes), lambda i: (i, 0)),
        out_shape=jax.ShapeDtypeStruct((B, n_nodes), jnp.float32),
        scratch_shapes=[pltpu.VMEM((BBB * n_nodes, d), jnp.bfloat16)],
        compiler_params=pltpu.CompilerParams(
            dimension_semantics=("parallel",)),
    )(y, y, bn_gamma.reshape(1, d), bn_beta.reshape(1, d), h,
      w_co[:, :d].astype(jnp.bfloat16), w_co[:, d:].astype(jnp.bfloat16),
      b_co, static_adj, dynamic_weight.astype(jnp.bfloat16), w_last)
    out2 = out2b + o2a.reshape(B, n_nodes) + b_last.reshape(1, n_nodes)
    return out1, out2


# speedup vs baseline: 1.0079x; 1.0079x over previous
import jax
import jax.numpy as jnp
from jax import lax
from jax.experimental import pallas as pl
from jax.experimental.pallas import tpu as pltpu

NEG_SLOPE = 0.2
BN_EPS = 1e-5
BBA = 16
BBB = 32
NPAD = 128
ROWS = 56


def _leaky(x):
    return jnp.where(x >= 0, x, NEG_SLOPE * x)


def _kern_a(x_ref, wcat_ref, btr_ref, adjn_ref, ws_ref, wg_ref,
            bg_ref, wlast_ref, out1_ref, h_ref, y_ref, o2a_ref, xb_scr,
            vt_scr, t_scr, *, n_nodes, d, hw):
    cf = x_ref.shape[2]
    for i in range(BBA):
        xb_scr[i * ROWS:i * ROWS + hw, :] = x_ref[i]
        xb_scr[i * ROWS + hw:(i + 1) * ROWS, :] = \
            jnp.zeros((ROWS - hw, cf), jnp.bfloat16)
    xb = xb_scr[...]
    s_all = jnp.dot(xb, wcat_ref[...],
                    preferred_element_type=jnp.float32)
    sc_all = s_all[:, :n_nodes]
    xt_all = s_all[:, NPAD:] + btr_ref[...]
    rows = ROWS
    valid = lax.broadcasted_iota(jnp.int32, (rows, 1), 0) < hw
    for i in range(BBA):
        sc = sc_all[i * rows:(i + 1) * rows, :]
        out1_ref[i:i + 1, :] = jnp.max(
            jnp.where(valid, sc, -jnp.inf), axis=0, keepdims=True)
        mask = jnp.where(valid, jax.nn.sigmoid(sc), 0.0)
        xt = xt_all[i * rows:(i + 1) * rows, :]
        vt_i = lax.dot_general(mask.astype(jnp.bfloat16),
                               xt.astype(jnp.bfloat16),
                               (((0,), (0,)), ((), ())),
                               preferred_element_type=jnp.float32)
        vt_scr[i * n_nodes:(i + 1) * n_nodes, :] = \
            vt_i.astype(jnp.bfloat16)
        o2a_ref[i * n_nodes:(i + 1) * n_nodes, :] = \
            jnp.sum(wlast_ref[...] * vt_i, axis=1, keepdims=True)
    v_bf = vt_scr[...]
    for i in range(BBA):
        t_i = jnp.dot(adjn_ref[...],
                      v_bf[i * n_nodes:(i + 1) * n_nodes, :],
                      preferred_element_type=jnp.float32)
        t_scr[i * n_nodes:(i + 1) * n_nodes, :] = \
            _leaky(t_i).astype(jnp.bfloat16)
    h_all = v_bf.astype(jnp.float32) + jnp.dot(
        t_scr[...], ws_ref[...], preferred_element_type=jnp.float32)
    h_ref[...] = h_all.astype(jnp.bfloat16)
    xglb = jnp.mean(h_all.reshape(BBA, n_nodes, d), axis=1)
    y_ref[...] = lax.dot_general(xglb.astype(jnp.bfloat16), wg_ref[...],
                                 (((1,), (1,)), ((), ())),
                                 preferred_element_type=jnp.float32) \
        + bg_ref[...]


def _kern_b(yfull_ref, yblk_ref, gamma_ref, beta_ref, h_ref, wcog_ref,
            wcox_ref, bco_ref, sadj_ref, wdyn_ref, wlast_ref, out2_ref,
            t_scr, *, n_nodes, d):
    yf = yfull_ref[...]
    mu = jnp.mean(yf, axis=0, keepdims=True)
    var = jnp.mean((yf - mu) ** 2, axis=0, keepdims=True)
    g = _leaky((yblk_ref[...] - mu) * lax.rsqrt(var + BN_EPS)
               * gamma_ref[...] + beta_ref[...])
    tg = lax.dot_general(wcog_ref[...], g.astype(jnp.bfloat16),
                         (((1,), (1,)), ((), ())),
                         preferred_element_type=jnp.float32)
    tx_all = lax.dot_general(wcox_ref[...], h_ref[...],
                             (((1,), (1,)), ((), ())),
                             preferred_element_type=jnp.float32)
    for i in range(BBB):
        h_bf = h_ref[i * n_nodes:(i + 1) * n_nodes, :]
        tx = tx_all[:, i * n_nodes:(i + 1) * n_nodes]
        a = jax.nn.sigmoid(tx + tg[:, i:i + 1] + bco_ref[...])
        a = (a + sadj_ref[...]) * 0.5
        dv = lax.rsqrt(jnp.sum(a, axis=1, keepdims=True))
        m = (dv * h_bf.astype(jnp.float32)).astype(jnp.bfloat16)
        t_i = lax.dot_general(a.astype(jnp.bfloat16), m,
                              (((0,), (0,)), ((), ())),
                              preferred_element_type=jnp.float32)
        t_scr[i * n_nodes:(i + 1) * n_nodes, :] = \
            _leaky(dv * t_i).astype(jnp.bfloat16)
    z = _leaky(jnp.dot(t_scr[...], wdyn_ref[...],
                       preferred_element_type=jnp.float32))
    zw = z.reshape(BBB, n_nodes, d) * wlast_ref[...][None]
    out2_ref[...] = jnp.sum(zw, axis=2)


def kernel(x_feat, static_adj, static_weight, dynamic_weight, w_fc, w_tr,
           b_tr, w_g, b_g, bn_gamma, bn_beta, w_co, b_co, w_last, b_last):
    B, Cf, H, W = x_feat.shape
    n_nodes = w_fc.shape[0]
    d = w_tr.shape[0]
    hw = H * W

    xt2 = x_feat.reshape(B, Cf, hw).astype(jnp.bfloat16)
    xt2 = xt2.transpose(0, 2, 1)
    wcat_t = jnp.concatenate(
        [w_fc.T.astype(jnp.bfloat16),
         jnp.zeros((Cf, NPAD - n_nodes), jnp.bfloat16),
         w_tr.T.astype(jnp.bfloat16)], axis=1)
    A = static_adj
    dvec = jnp.sum(A, axis=1) ** -0.5
    adjn = (dvec[:, None] * A.T * dvec[None, :]).astype(jnp.bfloat16)

    out1, h, y, o2a = pl.pallas_call(
        lambda *refs: _kern_a(*refs, n_nodes=n_nodes, d=d, hw=hw),
        grid=(B // BBA,),
        in_specs=[
            pl.BlockSpec((BBA, hw, Cf), lambda i: (i, 0, 0)),
            pl.BlockSpec((Cf, NPAD + d), lambda i: (0, 0)),
            pl.BlockSpec((1, d), lambda i: (0, 0)),
            pl.BlockSpec((n_nodes, n_nodes), lambda i: (0, 0)),
            pl.BlockSpec((d, d), lambda i: (0, 0)),
            pl.BlockSpec((d, d), lambda i: (0, 0)),
            pl.BlockSpec((1, d), lambda i: (0, 0)),
            pl.BlockSpec((n_nodes, d), lambda i: (0, 0)),
        ],
        out_specs=[
            pl.BlockSpec((BBA, n_nodes), lambda i: (i, 0)),
            pl.BlockSpec((BBA * n_nodes, d), lambda i: (i, 0)),
            pl.BlockSpec((BBA, d), lambda i: (i, 0)),
            pl.BlockSpec((BBA * n_nodes, 1), lambda i: (i, 0)),
        ],
        out_shape=[
            jax.ShapeDtypeStruct((B, n_nodes), jnp.float32),
            jax.ShapeDtypeStruct((B * n_nodes, d), jnp.bfloat16),
            jax.ShapeDtypeStruct((B, d), jnp.float32),
            jax.ShapeDtypeStruct((B * n_nodes, 1), jnp.float32),
        ],
        scratch_shapes=[pltpu.VMEM((BBA * ROWS, Cf), jnp.bfloat16),
                        pltpu.VMEM((BBA * n_nodes, d), jnp.bfloat16),
                        pltpu.VMEM((BBA * n_nodes, d), jnp.bfloat16)],
        compiler_params=pltpu.CompilerParams(
            dimension_semantics=("parallel",)),
    )(xt2, wcat_t, b_tr.reshape(1, d), adjn,
      static_weight.astype(jnp.bfloat16), w_g.astype(jnp.bfloat16),
      b_g.reshape(1, d), w_last)

    out2b = pl.pallas_call(
        lambda *refs: _kern_b(*refs, n_nodes=n_nodes, d=d),
        grid=(B // BBB,),
        in_specs=[
            pl.BlockSpec((B, d), lambda i: (0, 0)),
            pl.BlockSpec((BBB, d), lambda i: (i, 0)),
            pl.BlockSpec((1, d), lambda i: (0, 0)),
            pl.BlockSpec((1, d), lambda i: (0, 0)),
            pl.BlockSpec((BBB * n_nodes, d), lambda i: (i, 0)),
            pl.BlockSpec((n_nodes, d), lambda i: (0, 0)),
            pl.BlockSpec((n_nodes, d), lambda i: (0, 0)),
            pl.BlockSpec((n_nodes, 1), lambda i: (0, 0)),
            pl.BlockSpec((n_nodes, n_nodes), lambda i: (0, 0)),
            pl.BlockSpec((d, d), lambda i: (0, 0)),
            pl.BlockSpec((n_nodes, d), lambda i: (0, 0)),
        ],
        out_specs=pl.BlockSpec((BBB, n_nodes), lambda i: (i, 0)),
        out_shape=jax.ShapeDtypeStruct((B, n_nodes), jnp.float32),
        scratch_shapes=[pltpu.VMEM((BBB * n_nodes, d), jnp.bfloat16)],
        compiler_params=pltpu.CompilerParams(
            dimension_semantics=("parallel",)),
    )(y, y, bn_gamma.reshape(1, d), bn_beta.reshape(1, d), h,
      w_co[:, :d].astype(jnp.bfloat16), w_co[:, d:].astype(jnp.bfloat16),
      b_co, static_adj, dynamic_weight.astype(jnp.bfloat16), w_last)
    out2 = out2b + o2a.reshape(B, n_nodes) + b_last.reshape(1, n_nodes)
    return out1, out2
